# baseline (speedup 1.0000x reference)
import functools

import jax
import jax.numpy as jnp
from jax import lax
from jax.experimental import pallas as pl
from jax.experimental.pallas import tpu as pltpu
from jax.experimental.pallas import tpu_sc as plsc

_NC = 2
_NS = 16
_NW = _NC * _NS
_LANES = 16
_IDX_CHUNK = 128

_BATCH = 16384
_FACTORS = 32
_PACK = 128 // _FACTORS
_SUPER = _PACK * _FACTORS
_B_PER_W = _BATCH // _NW
_N_CHUNKS = _B_PER_W // _IDX_CHUNK
_GP_CHUNK = _IDX_CHUNK // _LANES


def _gmf_body(usup, isup, uqo, iqo, utab, itab, wv_hbm, bv_hbm, out,
              uidx, iidx, uq, iq, urows, irows, wv, bv, outv, sem):
    wid = lax.axis_index("s") * _NC + lax.axis_index("c")

    pltpu.sync_copy(usup.at[pl.ds(wid * _N_CHUNKS, _N_CHUNKS)], uidx)
    pltpu.sync_copy(isup.at[pl.ds(wid * _N_CHUNKS, _N_CHUNKS)], iidx)
    pltpu.sync_copy(uqo.at[pl.ds(wid * _B_PER_W, _B_PER_W)], uq)
    pltpu.sync_copy(iqo.at[pl.ds(wid * _B_PER_W, _B_PER_W)], iq)
    pltpu.sync_copy(wv_hbm, wv)
    pltpu.sync_copy(bv_hbm, bv)

    w_lo = wv[pl.ds(0, _LANES)]
    w_hi = wv[pl.ds(_LANES, _LANES)]
    bias = bv[...]
    lane = lax.iota(jnp.int32, _LANES)

    def half(h, carry):
        hbase = h * (_B_PER_W // 2)
        copies = []
        for c2 in range(_N_CHUNKS // 2):
            c = h * (_N_CHUNKS // 2) + c2
            sl = pl.ds(c2 * _IDX_CHUNK, _IDX_CHUNK)
            copies.append(
                pltpu.async_copy(utab.at[uidx.at[c]], urows.at[sl], sem))
            copies.append(
                pltpu.async_copy(itab.at[iidx.at[c]], irows.at[sl], sem))
        for cp in copies:
            cp.wait()

        def group(g, carry2):
            lbase = g * _LANES
            base = hbase + lbase
            uqv = uq[pl.ds(base, _LANES)]
            iqv = iq[pl.ds(base, _LANES)]
            acc = jnp.zeros((_LANES,), jnp.float32)
            for r in range(_LANES):
                row = lbase + r
                qu = uqv[r]
                qi = iqv[r]
                u_lo = urows[row, pl.ds(qu, _LANES)]
                u_hi = urows[row, pl.ds(qu + _LANES, _LANES)]
                i_lo = irows[row, pl.ds(qi, _LANES)]
                i_hi = irows[row, pl.ds(qi + _LANES, _LANES)]
                p = u_lo * i_lo * w_lo + u_hi * i_hi * w_hi
                acc = jnp.where(lane == r, jnp.sum(p), acc)
            logits = acc + bias
            outv[pl.ds(base, _LANES)] = 1.0 / (1.0 + jnp.exp(-logits))
            return carry2

        lax.fori_loop(0, _B_PER_W // 2 // _LANES, group, 0, unroll=False)
        return carry

    lax.fori_loop(0, 2, half, 0, unroll=False)

    pltpu.sync_copy(outv, out.at[pl.ds(wid * _B_PER_W, _B_PER_W)])


_gmf_call = functools.partial(
    pl.kernel,
    out_type=jax.ShapeDtypeStruct((_BATCH,), jnp.float32),
    mesh=plsc.VectorSubcoreMesh(core_axis_name="c", subcore_axis_name="s"),
    compiler_params=pltpu.CompilerParams(
        needs_layout_passes=False, use_tc_tiling_on_sc=True),
    scratch_types=[
        pltpu.VMEM((_N_CHUNKS, _IDX_CHUNK), jnp.int32),
        pltpu.VMEM((_N_CHUNKS, _IDX_CHUNK), jnp.int32),
        pltpu.VMEM((_B_PER_W,), jnp.int32),
        pltpu.VMEM((_B_PER_W,), jnp.int32),
        pltpu.VMEM((_B_PER_W // 2, _SUPER), jnp.float32),
        pltpu.VMEM((_B_PER_W // 2, _SUPER), jnp.float32),
        pltpu.VMEM((_FACTORS,), jnp.float32),
        pltpu.VMEM((_LANES,), jnp.float32),
        pltpu.VMEM((_B_PER_W,), jnp.float32),
        pltpu.SemaphoreType.DMA,
    ],
)(_gmf_body)


@jax.jit
def kernel(user_ids, item_ids, user_table, item_table, W, b):
    uids = user_ids.astype(jnp.int32)
    iids = item_ids.astype(jnp.int32)
    usup = (uids // _PACK).reshape(_NW * _N_CHUNKS, _IDX_CHUNK)
    isup = (iids // _PACK).reshape(_NW * _N_CHUNKS, _IDX_CHUNK)
    uqo = (uids % _PACK) * _FACTORS
    iqo = (iids % _PACK) * _FACTORS
    n_rows = user_table.shape[0]
    one = (b * 0.0 + 1.0).astype(jnp.float32)[0]
    ut4 = user_table.reshape(n_rows // _PACK, _SUPER) * one
    it4 = item_table.reshape(n_rows // _PACK, _SUPER) * one
    wv = W.reshape(_FACTORS).astype(jnp.float32)
    bv = jnp.broadcast_to(b.reshape(()), (_LANES,)).astype(jnp.float32)
    out = _gmf_call(usup, isup, uqo, iqo, ut4, it4, wv, bv)
    return out.reshape(_BATCH, 1)


# speedup vs baseline: 2.7863x; 2.7863x over previous
import functools

import jax
import jax.numpy as jnp
from jax import lax
from jax.experimental import pallas as pl
from jax.experimental.pallas import tpu as pltpu
from jax.experimental.pallas import tpu_sc as plsc

_NC = 2
_NS = 16
_NW = _NC * _NS
_LANES = 16
_IDX_CHUNK = 128

_BATCH = 16384
_FACTORS = 32
_PACK = 128 // _FACTORS
_SUPER = _PACK * _FACTORS
_B_PER_W = _BATCH // _NW
_N_CHUNKS = _B_PER_W // _IDX_CHUNK
_GP_CHUNK = _IDX_CHUNK // _LANES


def _gmf_body(usup, isup, uqo, iqo, utab, itab, wv_hbm, bv_hbm, out,
              uidx, iidx, uq, iq, urows, irows, wv, bv, outv, sem):
    wid = lax.axis_index("s") * _NC + lax.axis_index("c")

    pltpu.sync_copy(usup.at[pl.ds(wid * _N_CHUNKS, _N_CHUNKS)], uidx)
    pltpu.sync_copy(isup.at[pl.ds(wid * _N_CHUNKS, _N_CHUNKS)], iidx)
    pltpu.sync_copy(uqo.at[pl.ds(wid * _B_PER_W, _B_PER_W)], uq)
    pltpu.sync_copy(iqo.at[pl.ds(wid * _B_PER_W, _B_PER_W)], iq)
    pltpu.sync_copy(wv_hbm, wv)
    pltpu.sync_copy(bv_hbm, bv)

    w_lo = wv[pl.ds(0, _LANES)]
    w_hi = wv[pl.ds(_LANES, _LANES)]
    bias = bv[...]
    lane = lax.iota(jnp.int32, _LANES)

    def half(h, carry):
        hbase = h * (_B_PER_W // 2)
        copies = []
        for c2 in range(_N_CHUNKS // 2):
            c = h * (_N_CHUNKS // 2) + c2
            sl = pl.ds(c2 * _IDX_CHUNK, _IDX_CHUNK)
            copies.append(
                pltpu.async_copy(utab.at[uidx.at[c]], urows.at[sl], sem))
            copies.append(
                pltpu.async_copy(itab.at[iidx.at[c]], irows.at[sl], sem))
        for cp in copies:
            cp.wait()

        def group(g, carry2):
            lbase = g * _LANES
            base = hbase + lbase
            uqv = uq[pl.ds(base, _LANES)]
            iqv = iq[pl.ds(base, _LANES)]
            acc = jnp.zeros((_LANES,), jnp.float32)
            for r in range(_LANES):
                row = lbase + r
                qu = uqv[r]
                qi = iqv[r]
                u_lo = urows[row, pl.ds(qu, _LANES)]
                u_hi = urows[row, pl.ds(qu + _LANES, _LANES)]
                i_lo = irows[row, pl.ds(qi, _LANES)]
                i_hi = irows[row, pl.ds(qi + _LANES, _LANES)]
                p = u_lo * i_lo * w_lo + u_hi * i_hi * w_hi
                acc = jnp.where(lane == r, jnp.sum(p), acc)
            logits = acc + bias
            outv[pl.ds(base, _LANES)] = 1.0 / (1.0 + jnp.exp(-logits))
            return carry2

        lax.fori_loop(0, _B_PER_W // 2 // _LANES, group, 0, unroll=False)
        return carry

    lax.fori_loop(0, 2, half, 0, unroll=False)

    pltpu.sync_copy(outv, out.at[pl.ds(wid * _B_PER_W, _B_PER_W)])


_gmf_call = functools.partial(
    pl.kernel,
    out_type=jax.ShapeDtypeStruct((_BATCH,), jnp.float32),
    mesh=plsc.VectorSubcoreMesh(core_axis_name="c", subcore_axis_name="s"),
    compiler_params=pltpu.CompilerParams(
        needs_layout_passes=False, use_tc_tiling_on_sc=True),
    scratch_types=[
        pltpu.VMEM((_N_CHUNKS, _IDX_CHUNK), jnp.int32),
        pltpu.VMEM((_N_CHUNKS, _IDX_CHUNK), jnp.int32),
        pltpu.VMEM((_B_PER_W,), jnp.int32),
        pltpu.VMEM((_B_PER_W,), jnp.int32),
        pltpu.VMEM((_B_PER_W // 2, _SUPER), jnp.float32),
        pltpu.VMEM((_B_PER_W // 2, _SUPER), jnp.float32),
        pltpu.VMEM((_FACTORS,), jnp.float32),
        pltpu.VMEM((_LANES,), jnp.float32),
        pltpu.VMEM((_B_PER_W,), jnp.float32),
        pltpu.SemaphoreType.DMA,
    ],
)(_gmf_body)


_N_TAB = 1000000
_UBLK = 16384
_QBLK = _UBLK // _PACK
_N_PBLK = -(-_N_TAB // _UBLK)
_SROWS = _N_PBLK * _QBLK


def _pack_body(u_ref, i_ref, uo_ref, io_ref):
    for src, dst in ((u_ref, uo_ref), (i_ref, io_ref)):
        dst[...] = jnp.concatenate(
            [src[:, q * _QBLK:(q + 1) * _QBLK].T for q in range(_PACK)],
            axis=1)


_pack_tables = pl.pallas_call(
    _pack_body,
    grid=(_N_PBLK,),
    in_specs=[
        pl.BlockSpec((_FACTORS, _UBLK), lambda i: (0, i)),
        pl.BlockSpec((_FACTORS, _UBLK), lambda i: (0, i)),
    ],
    out_specs=[
        pl.BlockSpec((_QBLK, _SUPER), lambda i: (i, 0)),
        pl.BlockSpec((_QBLK, _SUPER), lambda i: (i, 0)),
    ],
    out_shape=[
        jax.ShapeDtypeStruct((_SROWS, _SUPER), jnp.float32),
        jax.ShapeDtypeStruct((_SROWS, _SUPER), jnp.float32),
    ],
)


@jax.jit
def kernel(user_ids, item_ids, user_table, item_table, W, b):
    uids = user_ids.astype(jnp.int32)
    iids = item_ids.astype(jnp.int32)
    usup = ((uids // _UBLK) * _QBLK + uids % _QBLK
            ).reshape(_NW * _N_CHUNKS, _IDX_CHUNK)
    isup = ((iids // _UBLK) * _QBLK + iids % _QBLK
            ).reshape(_NW * _N_CHUNKS, _IDX_CHUNK)
    uqo = ((uids % _UBLK) // _QBLK) * _FACTORS
    iqo = ((iids % _UBLK) // _QBLK) * _FACTORS
    ut4, it4 = _pack_tables(user_table.T, item_table.T)
    wv = W.reshape(_FACTORS).astype(jnp.float32)
    bv = jnp.broadcast_to(b.reshape(()), (_LANES,)).astype(jnp.float32)
    out = _gmf_call(usup, isup, uqo, iqo, ut4, it4, wv, bv)
    return out.reshape(_BATCH, 1)
